# baseline (speedup 1.0000x reference)
import functools

import jax
import jax.numpy as jnp
from jax import lax
from jax.experimental import pallas as pl
from jax.experimental.pallas import tpu as pltpu
from jax.experimental.pallas import tpu_sc as plsc

_NUM_CLASSES = 21
_D = 8732
_BATCH = 8
_A = 20
_MATCH_THRESH = 0.5

_DP = 9216
_R, _C = 8, 1152
_NW = 32
_CHUNK = _DP // _NW
_NCH = _CHUNK // 16


def _sc_match_body(dcx_hbm, dcy_hbm, dw_hbm, dh_hbm, ann_hbm,
                   best_hbm, bwa_hbm, pmax_hbm, pidx_hbm,
                   dcx_v, dcy_v, dw_v, dh_v, annv, best_v, bwa_v,
                   pm_v, pi_v):
    f32 = jnp.float32
    wid = lax.axis_index("c") * 16 + lax.axis_index("s")
    base = wid * _CHUNK

    pltpu.sync_copy(dcx_hbm.at[pl.ds(base, _CHUNK)], dcx_v)
    pltpu.sync_copy(dcy_hbm.at[pl.ds(base, _CHUNK)], dcy_v)
    pltpu.sync_copy(dw_hbm.at[pl.ds(base, _CHUNK)], dw_v)
    pltpu.sync_copy(dh_hbm.at[pl.ds(base, _CHUNK)], dh_v)
    pltpu.sync_copy(ann_hbm, annv)

    alxs, alys, arxs, arys, aareas = [], [], [], [], []
    for h in range(2):
        acx = annv[pl.ds(h * 16, 16)]
        acy = annv[pl.ds(32 + h * 16, 16)]
        aw = annv[pl.ds(64 + h * 16, 16)]
        ah = annv[pl.ds(96 + h * 16, 16)]
        alx = jnp.maximum(acx - aw * 0.5, 0.0)
        aly = jnp.maximum(acy - ah * 0.5, 0.0)
        arx = jnp.minimum(acx + aw * 0.5, 1.0)
        ary = jnp.minimum(acy + ah * 0.5, 1.0)
        alxs.append(alx)
        alys.append(aly)
        arxs.append(arx)
        arys.append(ary)
        aareas.append((arx - alx) * (ary - aly))

    zeros = jnp.zeros((16,), f32)
    neg1 = jnp.full((16,), -1.0, f32)
    for a in range(_A):
        pm_v[pl.ds(a * 16, 16)] = neg1
        pi_v[pl.ds(a * 16, 16)] = zeros

    lane_f = lax.iota(jnp.int32, 16).astype(f32)
    base_f = base.astype(f32)

    def chunk_body(i, carry):
        off = i * 16
        s = pl.ds(off, 16)
        dcx = dcx_v[s]
        dcy = dcy_v[s]
        dw = dw_v[s]
        dh = dh_v[s]
        dlx = jnp.maximum(dcx - dw * 0.5, 0.0)
        dly = jnp.maximum(dcy - dh * 0.5, 0.0)
        drx = jnp.minimum(dcx + dw * 0.5, 1.0)
        dry = jnp.minimum(dcy + dh * 0.5, 1.0)
        darea = (drx - dlx) * (dry - dly)
        idxv = base_f + off.astype(f32) + lane_f

        best = neg1
        bwav = zeros
        for a in range(_A):
            h, l = divmod(a, 16)
            alx = alxs[h][l]
            aly = alys[h][l]
            arx = arxs[h][l]
            ary = arys[h][l]
            aarea = aareas[h][l]
            ix = jnp.maximum(jnp.minimum(drx, arx) - jnp.maximum(dlx, alx),
                             0.0)
            iy = jnp.maximum(jnp.minimum(dry, ary) - jnp.maximum(dly, aly),
                             0.0)
            inter = ix * iy
            iou = inter / (darea + aarea - inter + 1e-10)
            upd = iou > best
            bwav = jnp.where(upd, f32(a), bwav)
            best = jnp.where(upd, iou, best)
            sa = pl.ds(a * 16, 16)
            pm = pm_v[sa]
            pu = iou > pm
            pm_v[sa] = jnp.where(pu, iou, pm)
            pi_v[sa] = jnp.where(pu, idxv, pi_v[sa])
        best_v[s] = best
        bwa_v[s] = bwav
        return carry

    lax.fori_loop(0, _NCH, chunk_body, 0)

    pltpu.sync_copy(best_v, best_hbm.at[pl.ds(base, _CHUNK)])
    pltpu.sync_copy(bwa_v, bwa_hbm.at[pl.ds(base, _CHUNK)])
    pltpu.sync_copy(pm_v, pmax_hbm.at[pl.ds(wid * (_A * 16), _A * 16)])
    pltpu.sync_copy(pi_v, pidx_hbm.at[pl.ds(wid * (_A * 16), _A * 16)])


_sc_match = functools.partial(
    pl.kernel,
    out_type=[
        jax.ShapeDtypeStruct((_DP,), jnp.float32),
        jax.ShapeDtypeStruct((_DP,), jnp.float32),
        jax.ShapeDtypeStruct((_NW * _A * 16,), jnp.float32),
        jax.ShapeDtypeStruct((_NW * _A * 16,), jnp.float32),
    ],
    mesh=plsc.VectorSubcoreMesh(core_axis_name="c", subcore_axis_name="s"),
    scratch_types=[
        pltpu.VMEM((_CHUNK,), jnp.float32),
        pltpu.VMEM((_CHUNK,), jnp.float32),
        pltpu.VMEM((_CHUNK,), jnp.float32),
        pltpu.VMEM((_CHUNK,), jnp.float32),
        pltpu.VMEM((128,), jnp.float32),
        pltpu.VMEM((_CHUNK,), jnp.float32),
        pltpu.VMEM((_CHUNK,), jnp.float32),
        pltpu.VMEM((_A * 16,), jnp.float32),
        pltpu.VMEM((_A * 16,), jnp.float32),
    ],
)(_sc_match_body)


def _loss_kernel(ann_ref, db_ref, best_ref, bwa_ref, pmax_ref, pidx_ref,
                 pc_ref, po_ref, out_ref, matched_s, ccol_s, offs_s, acc_s):
    f32 = jnp.float32
    shp = (_R, _C)
    j = pl.program_id(0)

    @pl.when(j == 0)
    def _matching():
        _do_matching(ann_ref, db_ref, best_ref, bwa_ref, pmax_ref, pidx_ref,
                     matched_s, ccol_s, offs_s, acc_s)

    matched_f = matched_s[...]
    ccol = ccol_s[...]

    rows = [pc_ref[0, c] for c in range(_NUM_CLASSES)]
    m = rows[0]
    for c in range(1, _NUM_CLASSES):
        m = jnp.maximum(m, rows[c])
    z = jnp.zeros(shp, f32)
    e_true = jnp.zeros(shp, f32)
    for c in range(_NUM_CLASSES):
        e = jnp.exp(rows[c] - m)
        z = z + e
        e_true = jnp.where(ccol == f32(c), e, e_true)
    p = e_true / z
    p = jnp.clip(p, 1e-07, 1.0 - 1e-07)
    fl = -0.25 * jnp.log(p) * (1.0 - p) * (1.0 - p)
    step_sum = jnp.sum(fl * matched_f)
    reg = jnp.zeros(shp, f32)
    for k in range(4):
        d = po_ref[0, k] - offs_s[k]
        ad = jnp.abs(d)
        sl1 = jnp.where(ad < 1.0, 0.5 * d * d, ad - 0.5)
        reg = reg + sl1
    step_reg = jnp.sum(reg * matched_f)
    acc_s[0] = acc_s[0] + step_sum
    acc_s[1] = acc_s[1] + step_reg

    @pl.when(j == _BATCH - 1)
    def _final():
        inv = 1.0 / (f32(_BATCH) * acc_s[2])
        cls_loss = acc_s[0] * inv
        reg_loss = acc_s[1] * inv
        r_iota = lax.broadcasted_iota(jnp.int32, (8, 128), 0)
        l_iota = lax.broadcasted_iota(jnp.int32, (8, 128), 1)
        vals = jnp.where(l_iota == 0, cls_loss + reg_loss,
                         jnp.where(l_iota == 1, reg_loss,
                                   jnp.where(l_iota == 2, cls_loss, 0.0)))
        out_ref[:, :] = jnp.where(r_iota == 0, vals, 0.0)


def _do_matching(ann_ref, db_ref, best_ref, bwa_ref, pmax_ref, pidx_ref,
                 matched_s, ccol_s, offs_s, acc_s):
    f32 = jnp.float32
    shp = (_R, _C)
    d_iota = (lax.broadcasted_iota(jnp.int32, shp, 0) * _C
              + lax.broadcasted_iota(jnp.int32, shp, 1)).astype(f32)

    awb = []
    for a in range(_A):
        v = pmax_ref[pl.ds(a, 1), :]
        i = pidx_ref[pl.ds(a, 1), :]
        mv = jnp.max(v)
        awb.append(jnp.min(jnp.where(v == mv, i, f32(1e9))))

    best = best_ref[...]
    bwa = bwa_ref[...]
    matched = best >= _MATCH_THRESH
    for a in range(_A):
        hit = d_iota == awb[a]
        matched = jnp.logical_or(matched, hit)
        bwa = jnp.where(hit, f32(a), bwa)

    matched_f = matched.astype(f32)
    acc_s[0] = f32(0.0)
    acc_s[1] = f32(0.0)
    acc_s[2] = jnp.sum(matched_f)

    tcx = jnp.zeros(shp, f32)
    tcy = jnp.zeros(shp, f32)
    tw = jnp.zeros(shp, f32)
    th = jnp.zeros(shp, f32)
    tcls = jnp.zeros(shp, f32)
    for a in range(_A):
        sel = bwa == f32(a)
        tcx = jnp.where(sel, ann_ref[a, 2], tcx)
        tcy = jnp.where(sel, ann_ref[a, 3], tcy)
        tw = jnp.where(sel, ann_ref[a, 4], tw)
        th = jnp.where(sel, ann_ref[a, 5], th)
        tcls = jnp.where(sel, ann_ref[a, 1], tcls)

    dcx = db_ref[0]
    dcy = db_ref[1]
    dw = db_ref[2]
    dh = db_ref[3]
    safe_w = jnp.where(dw > 0.0, dw, 1.0)
    safe_h = jnp.where(dh > 0.0, dh, 1.0)
    offs_s[0] = (tcx - dcx) / (safe_w * 0.1)
    offs_s[1] = (tcy - dcy) / (safe_h * 0.1)
    offs_s[2] = jnp.log(jnp.where(tw > 0.0, tw, 1.0) / safe_w) * 5.0
    offs_s[3] = jnp.log(jnp.where(th > 0.0, th, 1.0) / safe_h) * 5.0

    ccol_s[...] = jnp.where(matched, tcls - 1.0, f32(-1.0))
    matched_s[...] = matched_f


@jax.jit
def kernel(predicted_offsets, predicted_classes, targets, default_boxes):
    f32 = jnp.float32
    ann = jnp.pad(targets[:_A], ((0, 0), (0, 2))).astype(f32)

    db_p = jnp.pad(default_boxes, ((0, _DP - _D), (0, 0)))
    ann_sc = jnp.pad(jnp.transpose(targets[:_A, 2:6]),
                     ((0, 0), (0, 32 - _A))).astype(f32).reshape(-1)

    best, bwa, pmax, pidx = _sc_match(
        db_p[:, 0], db_p[:, 1], db_p[:, 2], db_p[:, 3], ann_sc)
    best_t = best.reshape(_R, _C)
    bwa_t = bwa.reshape(_R, _C)
    pmax_t = jnp.pad(
        jnp.transpose(pmax.reshape(_NW, _A, 16), (1, 0, 2)).reshape(
            _A, _NW * 16), ((0, 4), (0, 0)))
    pidx_t = jnp.pad(
        jnp.transpose(pidx.reshape(_NW, _A, 16), (1, 0, 2)).reshape(
            _A, _NW * 16), ((0, 4), (0, 0)))

    db_t = jnp.transpose(db_p.reshape(_R, _C, 4), (2, 0, 1))

    pc_t = jnp.pad(predicted_classes, ((0, 0), (0, _DP - _D), (0, 0)))
    pc_t = jnp.transpose(pc_t.reshape(_BATCH, _R, _C, _NUM_CLASSES),
                         (0, 3, 1, 2))

    po_t = jnp.pad(predicted_offsets, ((0, 0), (0, _DP - _D), (0, 0)))
    po_t = jnp.transpose(po_t.reshape(_BATCH, _R, _C, 4),
                         (0, 3, 1, 2))

    out = pl.pallas_call(
        _loss_kernel,
        grid=(_BATCH,),
        out_shape=jax.ShapeDtypeStruct((8, 128), f32),
        in_specs=[
            pl.BlockSpec(memory_space=pltpu.SMEM),
            pl.BlockSpec((4, _R, _C), lambda j: (0, 0, 0)),
            pl.BlockSpec((_R, _C), lambda j: (0, 0)),
            pl.BlockSpec((_R, _C), lambda j: (0, 0)),
            pl.BlockSpec((24, 512), lambda j: (0, 0)),
            pl.BlockSpec((24, 512), lambda j: (0, 0)),
            pl.BlockSpec((1, _NUM_CLASSES, _R, _C), lambda j: (j, 0, 0, 0)),
            pl.BlockSpec((1, 4, _R, _C), lambda j: (j, 0, 0, 0)),
        ],
        out_specs=pl.BlockSpec((8, 128), lambda j: (0, 0)),
        scratch_shapes=[
            pltpu.VMEM((_R, _C), f32),
            pltpu.VMEM((_R, _C), f32),
            pltpu.VMEM((4, _R, _C), f32),
            pltpu.SMEM((3,), f32),
        ],
    )(ann, db_t, best_t, bwa_t, pmax_t, pidx_t, pc_t, po_t)

    total = out[0, 0]
    reg_loss = out[0, 1]
    cls_loss = out[0, 2]
    return (total, reg_loss, cls_loss)
